# baseline (speedup 1.0000x reference)
import functools

import jax
import jax.numpy as jnp
from jax import lax
from jax.experimental import pallas as pl
from jax.experimental.pallas import tpu as pltpu
from jax.experimental.pallas import tpu_sc as plsc

NC = 2
NS = 16
NW = NC * NS
CHUNK = 128


def kernel(table0, table1, table2, table3, indexes):
    B = indexes.shape[0]
    D3 = table3.shape[1]
    OUT_D = 128
    bpw = B // NW
    nch = bpw // CHUNK

    idxr = indexes.astype(jnp.int32).reshape(NW, nch, CHUNK)
    fused = jnp.concatenate(
        [table0, table1, table2,
         jnp.pad(table3, ((0, 0), (0, OUT_D - 96 - D3)))], axis=1)

    mesh = plsc.VectorSubcoreMesh(core_axis_name="c", subcore_axis_name="s")

    @functools.partial(
        pl.kernel,
        mesh=mesh,
        out_type=jax.ShapeDtypeStruct((B, OUT_D), jnp.float32),
        compiler_params=pltpu.CompilerParams(
            use_tc_tiling_on_sc=False, needs_layout_passes=False),
        scratch_types=[
            pltpu.VMEM((nch, CHUNK), jnp.int32),
            pltpu.VMEM((bpw, OUT_D), jnp.float32),
            pltpu.SemaphoreType.DMA,
        ],
    )
    def sc_kernel(tab, idx_hbm, out_hbm,
                  idx_v, obuf, sem):
        wid = lax.axis_index("s") * NC + lax.axis_index("c")
        base = wid * bpw
        pltpu.sync_copy(idx_hbm.at[wid], idx_v)
        cps = []
        for j in range(nch):
            rows = pl.ds(j * CHUNK, CHUNK)
            cps.append(pltpu.async_copy(tab.at[idx_v.at[j]], obuf.at[rows], sem))
        for c in cps:
            c.wait()
        pltpu.sync_copy(obuf, out_hbm.at[pl.ds(base, bpw), :])

    return sc_kernel(fused, idxr)
